# baseline (speedup 1.0000x reference)
import functools

import jax
import jax.numpy as jnp
from jax import lax
from jax.experimental import pallas as pl
from jax.experimental.pallas import tpu as pltpu
from jax.experimental.pallas import tpu_sc as plsc

VOCAB = 1000
VOCAB_PAD = 1024
EMBED_DIM = 32
BATCH = 16384
PAD_DIM = 128
CHUNK = 128
NBUF = 4


@functools.lru_cache(maxsize=None)
def _build():
    info = plsc.get_sparse_core_info()
    nc, ns = info.num_cores, info.num_subcores
    nw = nc * ns
    b_per_w = BATCH // nw
    rows_per_stager = VOCAB_PAD // ns

    mesh = plsc.VectorSubcoreMesh(core_axis_name="c", subcore_axis_name="s")

    @functools.partial(
        pl.kernel,
        mesh=mesh,
        out_type=jax.ShapeDtypeStruct((BATCH, PAD_DIM), jnp.float32),
        scratch_types=[
            pltpu.VMEM((b_per_w,), jnp.int32),
            pltpu.VMEM((NBUF, CHUNK, PAD_DIM), jnp.float32),
            pltpu.VMEM_SHARED((VOCAB_PAD, PAD_DIM), jnp.float32),
            pltpu.SemaphoreType.DMA,
            pltpu.SemaphoreType.DMA,
        ] + [pltpu.SemaphoreType.DMA] * NBUF,
    )
    def gather_kernel(idx_hbm, table_hbm, out_hbm, idx_v, rows_v, table_sp,
                      isem, gsem, *wsems):
        sid = lax.axis_index("s")
        wid = sid * nc + lax.axis_index("c")
        base = wid * b_per_w
        n_chunks = b_per_w // CHUNK
        idx_cp = pltpu.async_copy(
            idx_hbm.at[pl.ds(base, b_per_w)], idx_v, isem)
        pltpu.sync_copy(
            table_hbm.at[pl.ds(sid * rows_per_stager, rows_per_stager)],
            table_sp.at[pl.ds(sid * rows_per_stager, rows_per_stager)])
        idx_cp.wait()
        plsc.subcore_barrier()
        writes = [None] * NBUF
        for k in range(n_chunks):
            b = k % NBUF
            if writes[b] is not None:
                writes[b].wait()
            pltpu.async_copy(
                table_sp.at[idx_v.at[pl.ds(k * CHUNK, CHUNK)]],
                rows_v.at[b], gsem).wait()
            writes[b] = pltpu.async_copy(
                rows_v.at[b], out_hbm.at[pl.ds(base + k * CHUNK, CHUNK)],
                wsems[b])
        for w in writes:
            if w is not None:
                w.wait()

    return gather_kernel


@functools.lru_cache(maxsize=None)
def _build_transpose():
    blk = 512

    def body(in_ref, out_ref):
        t = lax.transpose(in_ref[...], (1, 0))
        out_ref[...] = t[:EMBED_DIM, :]

    return pl.pallas_call(
        body,
        grid=(BATCH // blk,),
        in_specs=[pl.BlockSpec((blk, PAD_DIM), lambda i: (i, 0))],
        out_specs=pl.BlockSpec((EMBED_DIM, blk), lambda i: (0, i)),
        out_shape=jax.ShapeDtypeStruct((EMBED_DIM, BATCH), jnp.float32),
    )


def kernel(user_id, table):
    table_padded = jnp.pad(
        table, ((0, VOCAB_PAD - VOCAB), (0, PAD_DIM - EMBED_DIM)))
    out_padded = _build()(user_id, table_padded)
    return _build_transpose()(out_padded).T


# speedup vs baseline: 1.4433x; 1.4433x over previous
import functools

import jax
import jax.numpy as jnp
from jax import lax
from jax.experimental import pallas as pl
from jax.experimental.pallas import tpu as pltpu
from jax.experimental.pallas import tpu_sc as plsc

VOCAB = 1000
VOCAB_PAD = 1024
EMBED_DIM = 32
BATCH = 16384
PAD_DIM = 128
CHUNK = 128
NBUF = 4


@functools.lru_cache(maxsize=None)
def _build():
    info = plsc.get_sparse_core_info()
    nc, ns = info.num_cores, info.num_subcores
    nw = nc * ns
    b_per_w = BATCH // nw
    rows_per_stager = VOCAB_PAD // ns

    mesh = plsc.VectorSubcoreMesh(core_axis_name="c", subcore_axis_name="s")

    @functools.partial(
        pl.kernel,
        mesh=mesh,
        out_type=jax.ShapeDtypeStruct((BATCH, PAD_DIM), jnp.float32),
        scratch_types=[
            pltpu.VMEM((b_per_w,), jnp.int32),
            pltpu.VMEM((NBUF, CHUNK, PAD_DIM), jnp.float32),
            pltpu.VMEM_SHARED((VOCAB_PAD, PAD_DIM), jnp.float32),
            pltpu.SemaphoreType.DMA,
            pltpu.SemaphoreType.DMA,
        ] + [pltpu.SemaphoreType.DMA] * NBUF,
    )
    def gather_kernel(idx_hbm, table_hbm, out_hbm, idx_v, rows_v, table_sp,
                      isem, gsem, *wsems):
        sid = lax.axis_index("s")
        wid = sid * nc + lax.axis_index("c")
        base = wid * b_per_w
        n_chunks = b_per_w // CHUNK
        idx_cp = pltpu.async_copy(
            idx_hbm.at[pl.ds(base, b_per_w)], idx_v, isem)
        pltpu.sync_copy(
            table_hbm.at[pl.ds(sid * rows_per_stager, rows_per_stager)],
            table_sp.at[pl.ds(sid * rows_per_stager, rows_per_stager)])
        idx_cp.wait()
        plsc.subcore_barrier()
        writes = [None] * NBUF
        for k in range(n_chunks):
            b = k % NBUF
            if writes[b] is not None:
                writes[b].wait()
            pltpu.async_copy(
                table_sp.at[idx_v.at[pl.ds(k * CHUNK, CHUNK)]],
                rows_v.at[b], gsem).wait()
            writes[b] = pltpu.async_copy(
                rows_v.at[b], out_hbm.at[pl.ds(base + k * CHUNK, CHUNK)],
                wsems[b])
        for w in writes:
            if w is not None:
                w.wait()

    return gather_kernel


def kernel(user_id, table):
    table_padded = jnp.pad(
        table, ((0, VOCAB_PAD - VOCAB), (0, PAD_DIM - EMBED_DIM)))
    out_padded = _build()(user_id, table_padded)
    return out_padded[:, :EMBED_DIM]
